# baseline (speedup 1.0000x reference)
import jax
import jax.numpy as jnp
from jax import lax
from jax.experimental import pallas as pl
from jax.experimental.pallas import tpu as pltpu
from jax.experimental.pallas import tpu_sc as plsc
import functools

NC = 2
NS = 16
CH = 128
TR = 128
SB = 16


def _sc_scatter(h, src2, dst2, zeros_agg, zeros_deg, half, d):
    nch = src2.shape[0]
    ch_per_tile = nch // NS
    hb = half + TR
    rows_per_tile = hb // NS
    cap = ch_per_tile * CH + 2 * CH

    mesh = plsc.VectorSubcoreMesh(
        core_axis_name="c", subcore_axis_name="s",
        num_cores=NC, num_subcores=NS)

    @functools.partial(
        pl.kernel,
        out_type=[
            jax.ShapeDtypeStruct((NC, hb, d), jnp.float32),
            jax.ShapeDtypeStruct((NS, NC * hb), jnp.float32),
        ],
        mesh=mesh,
        compiler_params=pltpu.CompilerParams(needs_layout_passes=False),
        scratch_types=[
            pltpu.VMEM((SB, CH), jnp.int32),
            pltpu.VMEM((SB, CH), jnp.int32),
            pltpu.VMEM((cap,), jnp.int32),
            pltpu.VMEM((cap,), jnp.int32),
            pltpu.VMEM((CH,), jnp.int32),
            pltpu.VMEM((CH, d), jnp.float32),
            pltpu.VMEM((4 * hb,), jnp.float32),
            pltpu.VMEM((hb,), jnp.float32),
            pltpu.VMEM_SHARED((hb, d), jnp.float32),
            pltpu.SemaphoreType.DMA,
        ],
    )
    def k(h_hbm, src_hbm, dst_hbm, zagg_hbm, zdeg_hbm,
          oagg_hbm, odeg_hbm,
          sblk_v, dblk_v, csrc_v, cdst_v, ldst_v, rows_v,
          deg4_v, deg_v, agg_sh, semA):
        c = lax.axis_index("c")
        s = lax.axis_index("s")
        lo = c * half

        r0 = s * rows_per_tile
        pltpu.sync_copy(zagg_hbm.at[pl.ds(r0, rows_per_tile)],
                        agg_sh.at[pl.ds(r0, rows_per_tile)])
        pltpu.sync_copy(zdeg_hbm, deg4_v)
        plsc.subcore_barrier()

        iota = lax.iota(jnp.int32, 16)
        copy_idx = iota & 3
        ones16 = jnp.ones((16,), jnp.float32)
        zeros16i = jnp.zeros((16,), jnp.int32)
        trash16i = jnp.full((16,), half, jnp.int32)
        group_masks = [(iota >> 2) == g for g in range(4)]

        def fblock(b, cw):
            pltpu.sync_copy(
                src_hbm.at[pl.ds(s * ch_per_tile + b * SB, SB)], sblk_v)
            pltpu.sync_copy(
                dst_hbm.at[pl.ds(s * ch_per_tile + b * SB, SB)], dblk_v)

            def frow(r, cw):
                for v in range(CH // 16):
                    dv = dblk_v[r, pl.ds(v * 16, 16)]
                    sv = sblk_v[r, pl.ds(v * 16, 16)]
                    inr = (dv >= lo) & (dv < lo + half)
                    lv = dv - lo
                    plsc.store_compressed(csrc_v.at[pl.ds(cw, 16)], sv,
                                          mask=inr)
                    plsc.store_compressed(cdst_v.at[pl.ds(cw, 16)], lv,
                                          mask=inr)
                    fidx = copy_idx * hb + jnp.where(inr, lv, half)
                    for g in range(4):
                        plsc.addupdate_scatter(deg4_v, [fidx], ones16,
                                               mask=inr & group_masks[g])
                    cw = cw + jnp.sum(inr.astype(jnp.int32))
                return cw

            return lax.fori_loop(0, SB, frow, cw)

        cw = lax.fori_loop(0, ch_per_tile // SB, fblock, 0)

        for g2 in range(9):
            pidx = cw + g2 * 16 + iota
            plsc.store_scatter(csrc_v, [pidx], zeros16i)
            plsc.store_scatter(cdst_v, [pidx], trash16i)

        def body(j, _):
            cp = pltpu.async_copy(
                h_hbm.at[csrc_v.at[pl.ds(j * CH, CH)]], rows_v, semA)
            for v in range(CH // 16):
                ldst_v[pl.ds(v * 16, 16)] = cdst_v[pl.ds(j * CH + v * 16, 16)]
            cp.wait()
            pltpu.sync_copy(rows_v, agg_sh.at[ldst_v], add=True)
            return _

        nct = (cw + (CH - 1)) // CH
        lax.fori_loop(0, nct, body, None)

        def red(i, _):
            b = i * 16
            deg_v[pl.ds(b, 16)] = (
                (deg4_v[pl.ds(b, 16)] + deg4_v[pl.ds(hb + b, 16)])
                + (deg4_v[pl.ds(2 * hb + b, 16)]
                   + deg4_v[pl.ds(3 * hb + b, 16)]))
            return _

        lax.fori_loop(0, hb // 16, red, None)
        plsc.subcore_barrier()

        pltpu.sync_copy(agg_sh.at[pl.ds(r0, rows_per_tile)],
                        oagg_hbm.at[c, pl.ds(r0, rows_per_tile)])
        pltpu.sync_copy(deg_v, odeg_hbm.at[s, pl.ds(c * hb, hb)])

    return k(h, src2, dst2, zeros_agg, zeros_deg)


def _tc_mlp(pagg, pdeg, h, W1, b1, W2, b2, gamma, beta, n, half, d, out_d):
    hb = half + TR
    n1 = n - half

    def body(pa_ref, pd_ref, h_ref, w1_ref, b1_ref, w2_ref, b2_ref,
             g_ref, be_ref, o_ref):
        agg = jnp.concatenate([pa_ref[0, :half, :], pa_ref[1, :n1, :]], axis=0)
        ones_col = jnp.ones((pd_ref.shape[0], 1), jnp.float32)
        deg_col = lax.dot_general(pd_ref[...], ones_col,
                                  dimension_numbers=(((0,), (0,)), ((), ())),
                                  preferred_element_type=jnp.float32)
        deg = jnp.concatenate([deg_col[:half, :], deg_col[hb:hb + n1, :]],
                              axis=0)
        h_in = agg / jnp.maximum(deg, 1.0) + h_ref[...]
        z = jnp.dot(h_in, w1_ref[...], preferred_element_type=jnp.float32)
        z = jnp.maximum(z + b1_ref[...], 0.0)
        z = jnp.dot(z, w2_ref[...], preferred_element_type=jnp.float32)
        z = jnp.maximum(z + b2_ref[...], 0.0)
        mean = jnp.mean(z, axis=0, keepdims=True)
        zc = z - mean
        var = jnp.mean(zc * zc, axis=0, keepdims=True)
        o_ref[...] = zc * lax.rsqrt(var + 1e-5) * g_ref[...] + be_ref[...]

    return pl.pallas_call(
        body,
        out_shape=jax.ShapeDtypeStruct((n, out_d), jnp.float32),
    )(pagg, pdeg, h, W1, b1, W2, b2, gamma, beta)


def kernel(h, edge_index, W1, b1, W2, b2, gamma, beta):
    n, d = h.shape
    e = edge_index.shape[1]
    hdim = W1.shape[1]
    out_d = W2.shape[1]

    half = ((n + 2 * 128 - 1) // (2 * 128)) * 128
    epg = CH * NS * 8
    e_pad = ((e + epg - 1) // epg) * epg
    pad = e_pad - e

    src = edge_index[0].astype(jnp.int32)
    dst = edge_index[1].astype(jnp.int32)
    if pad:
        src = jnp.concatenate([src, jnp.zeros((pad,), jnp.int32)])
        dst = jnp.concatenate([dst, jnp.full((pad,), n, jnp.int32)])
    src2 = src.reshape(-1, CH)
    dst2 = dst.reshape(-1, CH)

    hb = half + TR
    zeros_agg = jnp.zeros((hb, d), jnp.float32)
    zeros_deg = jnp.zeros((4 * hb,), jnp.float32)

    pagg, pdeg = _sc_scatter(h, src2, dst2, zeros_agg, zeros_deg, half, d)
    return _tc_mlp(pagg, pdeg, h,
                   W1, b1.reshape(1, hdim), W2, b2.reshape(1, out_d),
                   gamma.reshape(1, out_d), beta.reshape(1, out_d),
                   n, half, d, out_d)


# speedup vs baseline: 1.0026x; 1.0026x over previous
import jax
import jax.numpy as jnp
from jax import lax
from jax.experimental import pallas as pl
from jax.experimental.pallas import tpu as pltpu
from jax.experimental.pallas import tpu_sc as plsc
import functools

NC = 2
NS = 16
CH = 128
TR = 128
SB = 16


def _sc_scatter(h, src2, dst2, zeros_agg, zeros_deg, half, d):
    nch = src2.shape[0]
    ch_per_tile = nch // NS
    hb = half + TR
    rows_per_tile = hb // NS
    cap = ch_per_tile * CH + 2 * CH

    mesh = plsc.VectorSubcoreMesh(
        core_axis_name="c", subcore_axis_name="s",
        num_cores=NC, num_subcores=NS)

    @functools.partial(
        pl.kernel,
        out_type=[
            jax.ShapeDtypeStruct((NC, hb, d), jnp.float32),
            jax.ShapeDtypeStruct((NS, NC * hb), jnp.float32),
        ],
        mesh=mesh,
        compiler_params=pltpu.CompilerParams(needs_layout_passes=False),
        scratch_types=[
            pltpu.VMEM((SB, CH), jnp.int32),
            pltpu.VMEM((SB, CH), jnp.int32),
            pltpu.VMEM((cap,), jnp.int32),
            pltpu.VMEM((cap,), jnp.int32),
            pltpu.VMEM((CH,), jnp.int32),
            pltpu.VMEM((CH, d), jnp.float32),
            pltpu.VMEM((4 * hb,), jnp.float32),
            pltpu.VMEM((hb,), jnp.float32),
            pltpu.VMEM_SHARED((hb, d), jnp.float32),
            pltpu.SemaphoreType.DMA,
        ],
    )
    def k(h_hbm, src_hbm, dst_hbm, zagg_hbm, zdeg_hbm,
          oagg_hbm, odeg_hbm,
          sblk_v, dblk_v, csrc_v, cdst_v, ldst_v, rows_v,
          deg4_v, deg_v, agg_sh, semA):
        c = lax.axis_index("c")
        s = lax.axis_index("s")
        lo = c * half

        r0 = s * rows_per_tile
        pltpu.sync_copy(zagg_hbm.at[pl.ds(r0, rows_per_tile)],
                        agg_sh.at[pl.ds(r0, rows_per_tile)])
        pltpu.sync_copy(zdeg_hbm, deg4_v)
        plsc.subcore_barrier()

        iota = lax.iota(jnp.int32, 16)
        copy_idx = iota & 3
        ones16 = jnp.ones((16,), jnp.float32)
        zeros16i = jnp.zeros((16,), jnp.int32)
        trash16i = jnp.full((16,), half, jnp.int32)
        group_masks = [(iota >> 2) == g for g in range(4)]

        def fblock(b, cw):
            pltpu.sync_copy(
                src_hbm.at[pl.ds(s * ch_per_tile + b * SB, SB)], sblk_v)
            pltpu.sync_copy(
                dst_hbm.at[pl.ds(s * ch_per_tile + b * SB, SB)], dblk_v)

            def frow(r, cw):
                for v in range(CH // 16):
                    dv = dblk_v[r, pl.ds(v * 16, 16)]
                    sv = sblk_v[r, pl.ds(v * 16, 16)]
                    inr = (dv >= lo) & (dv < lo + half)
                    lv = dv - lo
                    plsc.store_compressed(csrc_v.at[pl.ds(cw, 16)], sv,
                                          mask=inr)
                    plsc.store_compressed(cdst_v.at[pl.ds(cw, 16)], lv,
                                          mask=inr)
                    cw = cw + jnp.sum(inr.astype(jnp.int32))
                return cw

            return lax.fori_loop(0, SB, frow, cw)

        cw = lax.fori_loop(0, ch_per_tile // SB, fblock, 0)

        for g2 in range(9):
            pidx = cw + g2 * 16 + iota
            plsc.store_scatter(csrc_v, [pidx], zeros16i)
            plsc.store_scatter(cdst_v, [pidx], trash16i)

        def body(j, _):
            cp = pltpu.async_copy(
                h_hbm.at[csrc_v.at[pl.ds(j * CH, CH)]], rows_v, semA)
            for v in range(CH // 16):
                lv = cdst_v[pl.ds(j * CH + v * 16, 16)]
                ldst_v[pl.ds(v * 16, 16)] = lv
                fidx = copy_idx * hb + lv
                for g in range(4):
                    plsc.addupdate_scatter(deg4_v, [fidx], ones16,
                                           mask=group_masks[g])
            cp.wait()
            pltpu.sync_copy(rows_v, agg_sh.at[ldst_v], add=True)
            return _

        nct = (cw + (CH - 1)) // CH
        lax.fori_loop(0, nct, body, None)

        def red(i, _):
            b = i * 16
            deg_v[pl.ds(b, 16)] = (
                (deg4_v[pl.ds(b, 16)] + deg4_v[pl.ds(hb + b, 16)])
                + (deg4_v[pl.ds(2 * hb + b, 16)]
                   + deg4_v[pl.ds(3 * hb + b, 16)]))
            return _

        lax.fori_loop(0, hb // 16, red, None)
        plsc.subcore_barrier()

        pltpu.sync_copy(agg_sh.at[pl.ds(r0, rows_per_tile)],
                        oagg_hbm.at[c, pl.ds(r0, rows_per_tile)])
        pltpu.sync_copy(deg_v, odeg_hbm.at[s, pl.ds(c * hb, hb)])

    return k(h, src2, dst2, zeros_agg, zeros_deg)


def _tc_mlp(pagg, pdeg, h, W1, b1, W2, b2, gamma, beta, n, half, d, out_d):
    hb = half + TR
    n1 = n - half

    def body(pa_ref, pd_ref, h_ref, w1_ref, b1_ref, w2_ref, b2_ref,
             g_ref, be_ref, o_ref):
        agg = jnp.concatenate([pa_ref[0, :half, :], pa_ref[1, :n1, :]], axis=0)
        ones_col = jnp.ones((pd_ref.shape[0], 1), jnp.float32)
        deg_col = lax.dot_general(pd_ref[...], ones_col,
                                  dimension_numbers=(((0,), (0,)), ((), ())),
                                  preferred_element_type=jnp.float32)
        deg = jnp.concatenate([deg_col[:half, :], deg_col[hb:hb + n1, :]],
                              axis=0)
        h_in = agg / jnp.maximum(deg, 1.0) + h_ref[...]
        z = jnp.dot(h_in, w1_ref[...], preferred_element_type=jnp.float32)
        z = jnp.maximum(z + b1_ref[...], 0.0)
        z = jnp.dot(z, w2_ref[...], preferred_element_type=jnp.float32)
        z = jnp.maximum(z + b2_ref[...], 0.0)
        mean = jnp.mean(z, axis=0, keepdims=True)
        zc = z - mean
        var = jnp.mean(zc * zc, axis=0, keepdims=True)
        o_ref[...] = zc * lax.rsqrt(var + 1e-5) * g_ref[...] + be_ref[...]

    return pl.pallas_call(
        body,
        out_shape=jax.ShapeDtypeStruct((n, out_d), jnp.float32),
    )(pagg, pdeg, h, W1, b1, W2, b2, gamma, beta)


def kernel(h, edge_index, W1, b1, W2, b2, gamma, beta):
    n, d = h.shape
    e = edge_index.shape[1]
    hdim = W1.shape[1]
    out_d = W2.shape[1]

    half = ((n + 2 * 128 - 1) // (2 * 128)) * 128
    epg = CH * NS * 8
    e_pad = ((e + epg - 1) // epg) * epg
    pad = e_pad - e

    src = edge_index[0].astype(jnp.int32)
    dst = edge_index[1].astype(jnp.int32)
    if pad:
        src = jnp.concatenate([src, jnp.zeros((pad,), jnp.int32)])
        dst = jnp.concatenate([dst, jnp.full((pad,), n, jnp.int32)])
    src2 = src.reshape(-1, CH)
    dst2 = dst.reshape(-1, CH)

    hb = half + TR
    zeros_agg = jnp.zeros((hb, d), jnp.float32)
    zeros_deg = jnp.zeros((4 * hb,), jnp.float32)

    pagg, pdeg = _sc_scatter(h, src2, dst2, zeros_agg, zeros_deg, half, d)
    return _tc_mlp(pagg, pdeg, h,
                   W1, b1.reshape(1, hdim), W2, b2.reshape(1, out_d),
                   gamma.reshape(1, out_d), beta.reshape(1, out_d),
                   n, half, d, out_d)


# speedup vs baseline: 1.0049x; 1.0023x over previous
import jax
import jax.numpy as jnp
from jax import lax
from jax.experimental import pallas as pl
from jax.experimental.pallas import tpu as pltpu
from jax.experimental.pallas import tpu_sc as plsc
import functools

NC = 2
NS = 16
CH = 128
TR = 128
SB = 16


def _sc_scatter(h, src2, dst2, zeros_agg, zeros_deg, half, d):
    nch = src2.shape[0]
    ch_per_tile = nch // NS
    hb = half + TR
    rows_per_tile = hb // NS
    cap = ch_per_tile * CH + 4 * CH

    mesh = plsc.VectorSubcoreMesh(
        core_axis_name="c", subcore_axis_name="s",
        num_cores=NC, num_subcores=NS)

    @functools.partial(
        pl.kernel,
        out_type=[
            jax.ShapeDtypeStruct((NC, hb, d), jnp.float32),
            jax.ShapeDtypeStruct((NS, NC * hb), jnp.float32),
        ],
        mesh=mesh,
        compiler_params=pltpu.CompilerParams(needs_layout_passes=False),
        scratch_types=[
            pltpu.VMEM((SB, CH), jnp.int32),
            pltpu.VMEM((SB, CH), jnp.int32),
            pltpu.VMEM((cap,), jnp.int32),
            pltpu.VMEM((2, CH), jnp.int32),
            pltpu.VMEM((2, CH), jnp.int32),
            pltpu.VMEM((CH, d), jnp.float32),
            pltpu.VMEM((4 * hb,), jnp.float32),
            pltpu.VMEM((hb,), jnp.float32),
            pltpu.VMEM_SHARED((hb, d), jnp.float32),
            pltpu.SemaphoreType.DMA,
        ],
    )
    def k(h_hbm, src_hbm, dst_hbm, zagg_hbm, zdeg_hbm,
          oagg_hbm, odeg_hbm,
          sblk_v, dblk_v, cpk_v, lsrc_v, ldst_v, rows_v,
          deg4_v, deg_v, agg_sh, semA):
        c = lax.axis_index("c")
        s = lax.axis_index("s")
        lo = c * half

        r0 = s * rows_per_tile
        pltpu.sync_copy(zagg_hbm.at[pl.ds(r0, rows_per_tile)],
                        agg_sh.at[pl.ds(r0, rows_per_tile)])
        pltpu.sync_copy(zdeg_hbm, deg4_v)
        plsc.subcore_barrier()

        iota = lax.iota(jnp.int32, 16)
        copy_idx = iota & 3
        ones16 = jnp.ones((16,), jnp.float32)
        zeros16i = jnp.zeros((16,), jnp.int32)
        trash16i = jnp.full((16,), half, jnp.int32)
        group_masks = [(iota >> 2) == g for g in range(4)]

        def fblock(b, cw):
            pltpu.sync_copy(
                src_hbm.at[pl.ds(s * ch_per_tile + b * SB, SB)], sblk_v)
            pltpu.sync_copy(
                dst_hbm.at[pl.ds(s * ch_per_tile + b * SB, SB)], dblk_v)

            def frow(r, cw):
                for v in range(CH // 16):
                    dv = dblk_v[r, pl.ds(v * 16, 16)]
                    sv = sblk_v[r, pl.ds(v * 16, 16)]
                    inr = (dv >= lo) & (dv < lo + half)
                    pk = (sv << 14) + (dv - lo)
                    plsc.store_compressed(cpk_v.at[pl.ds(cw, 16)], pk,
                                          mask=inr)
                    cw = cw + jnp.sum(inr.astype(jnp.int32))
                return cw

            return lax.fori_loop(0, SB, frow, cw)

        cw = lax.fori_loop(0, ch_per_tile // SB, fblock, 0)

        for g2 in range(17):
            pidx = cw + g2 * 16 + iota
            plsc.store_scatter(cpk_v, [pidx], trash16i)

        def stage(j, p):
            for v in range(CH // 16):
                pk = cpk_v[pl.ds(j * CH + v * 16, 16)]
                lv = pk & 16383
                lsrc_v[p, pl.ds(v * 16, 16)] = pk >> 14
                ldst_v[p, pl.ds(v * 16, 16)] = lv
                fidx = copy_idx * hb + lv
                for g in range(4):
                    plsc.addupdate_scatter(deg4_v, [fidx], ones16,
                                           mask=group_masks[g])

        stage(0, 0)

        def body(j, _):
            p = j & 1
            cp = pltpu.async_copy(h_hbm.at[lsrc_v.at[p]], rows_v, semA)
            stage(j + 1, 1 - p)
            cp.wait()
            pltpu.sync_copy(rows_v, agg_sh.at[ldst_v.at[p]], add=True)
            return _

        nct = (cw + (CH - 1)) // CH
        lax.fori_loop(0, nct, body, None)

        def red(i, _):
            b = i * 16
            deg_v[pl.ds(b, 16)] = (
                (deg4_v[pl.ds(b, 16)] + deg4_v[pl.ds(hb + b, 16)])
                + (deg4_v[pl.ds(2 * hb + b, 16)]
                   + deg4_v[pl.ds(3 * hb + b, 16)]))
            return _

        lax.fori_loop(0, hb // 16, red, None)
        plsc.subcore_barrier()

        pltpu.sync_copy(agg_sh.at[pl.ds(r0, rows_per_tile)],
                        oagg_hbm.at[c, pl.ds(r0, rows_per_tile)])
        pltpu.sync_copy(deg_v, odeg_hbm.at[s, pl.ds(c * hb, hb)])

    return k(h, src2, dst2, zeros_agg, zeros_deg)


def _tc_mlp(pagg, pdeg, h, W1, b1, W2, b2, gamma, beta, n, half, d, out_d):
    hb = half + TR
    n1 = n - half

    def body(pa_ref, pd_ref, h_ref, w1_ref, b1_ref, w2_ref, b2_ref,
             g_ref, be_ref, o_ref):
        agg = jnp.concatenate([pa_ref[0, :half, :], pa_ref[1, :n1, :]], axis=0)
        ones_col = jnp.ones((pd_ref.shape[0], 1), jnp.float32)
        deg_col = lax.dot_general(pd_ref[...], ones_col,
                                  dimension_numbers=(((0,), (0,)), ((), ())),
                                  preferred_element_type=jnp.float32)
        deg = jnp.concatenate([deg_col[:half, :], deg_col[hb:hb + n1, :]],
                              axis=0)
        h_in = agg / jnp.maximum(deg, 1.0) + h_ref[...]
        z = jnp.dot(h_in, w1_ref[...], preferred_element_type=jnp.float32)
        z = jnp.maximum(z + b1_ref[...], 0.0)
        z = jnp.dot(z, w2_ref[...], preferred_element_type=jnp.float32)
        z = jnp.maximum(z + b2_ref[...], 0.0)
        mean = jnp.mean(z, axis=0, keepdims=True)
        zc = z - mean
        var = jnp.mean(zc * zc, axis=0, keepdims=True)
        o_ref[...] = zc * lax.rsqrt(var + 1e-5) * g_ref[...] + be_ref[...]

    return pl.pallas_call(
        body,
        out_shape=jax.ShapeDtypeStruct((n, out_d), jnp.float32),
    )(pagg, pdeg, h, W1, b1, W2, b2, gamma, beta)


def kernel(h, edge_index, W1, b1, W2, b2, gamma, beta):
    n, d = h.shape
    e = edge_index.shape[1]
    hdim = W1.shape[1]
    out_d = W2.shape[1]

    half = ((n + 2 * 128 - 1) // (2 * 128)) * 128
    epg = CH * NS * 8
    e_pad = ((e + epg - 1) // epg) * epg
    pad = e_pad - e

    src = edge_index[0].astype(jnp.int32)
    dst = edge_index[1].astype(jnp.int32)
    if pad:
        src = jnp.concatenate([src, jnp.zeros((pad,), jnp.int32)])
        dst = jnp.concatenate([dst, jnp.full((pad,), n, jnp.int32)])
    src2 = src.reshape(-1, CH)
    dst2 = dst.reshape(-1, CH)

    hb = half + TR
    zeros_agg = jnp.zeros((hb, d), jnp.float32)
    zeros_deg = jnp.zeros((4 * hb,), jnp.float32)

    pagg, pdeg = _sc_scatter(h, src2, dst2, zeros_agg, zeros_deg, half, d)
    return _tc_mlp(pagg, pdeg, h,
                   W1, b1.reshape(1, hdim), W2, b2.reshape(1, out_d),
                   gamma.reshape(1, out_d), beta.reshape(1, out_d),
                   n, half, d, out_d)


# speedup vs baseline: 1.0121x; 1.0072x over previous
import jax
import jax.numpy as jnp
from jax import lax
from jax.experimental import pallas as pl
from jax.experimental.pallas import tpu as pltpu
from jax.experimental.pallas import tpu_sc as plsc
import functools

NC = 2
NS = 16
CH = 128
TR = 128
SB = 32


def _sc_scatter(h, epk2, zeros_agg, zeros_deg, half, d):
    nch = epk2.shape[0]
    ch_per_tile = nch // NS
    hb = half + TR
    rows_per_tile = hb // NS
    cap = ch_per_tile * CH + 4 * CH

    mesh = plsc.VectorSubcoreMesh(
        core_axis_name="c", subcore_axis_name="s",
        num_cores=NC, num_subcores=NS)

    @functools.partial(
        pl.kernel,
        out_type=[
            jax.ShapeDtypeStruct((NC, hb, d), jnp.float32),
            jax.ShapeDtypeStruct((NS, NC * hb), jnp.float32),
        ],
        mesh=mesh,
        compiler_params=pltpu.CompilerParams(needs_layout_passes=False),
        scratch_types=[
            pltpu.VMEM((SB, CH), jnp.int32),
            pltpu.VMEM((cap,), jnp.int32),
            pltpu.VMEM((2, CH), jnp.int32),
            pltpu.VMEM((2, CH), jnp.int32),
            pltpu.VMEM((CH, d), jnp.float32),
            pltpu.VMEM((4 * hb,), jnp.float32),
            pltpu.VMEM((hb,), jnp.float32),
            pltpu.VMEM_SHARED((hb, d), jnp.float32),
            pltpu.SemaphoreType.DMA,
        ],
    )
    def k(h_hbm, epk_hbm, zagg_hbm, zdeg_hbm,
          oagg_hbm, odeg_hbm,
          eblk_v, cpk_v, lsrc_v, ldst_v, rows_v,
          deg4_v, deg_v, agg_sh, semA):
        c = lax.axis_index("c")
        s = lax.axis_index("s")
        lo = c * half

        r0 = s * rows_per_tile
        pltpu.sync_copy(zagg_hbm.at[pl.ds(r0, rows_per_tile)],
                        agg_sh.at[pl.ds(r0, rows_per_tile)])
        pltpu.sync_copy(zdeg_hbm, deg4_v)
        plsc.subcore_barrier()

        iota = lax.iota(jnp.int32, 16)
        copy_idx = iota & 3
        ones16 = jnp.ones((16,), jnp.float32)
        trash16i = jnp.full((16,), half, jnp.int32)
        group_masks = [(iota >> 2) == g for g in range(4)]

        def fblock(b, cw):
            pltpu.sync_copy(
                epk_hbm.at[pl.ds(s * ch_per_tile + b * SB, SB)], eblk_v)

            def frow(r, cw):
                for v in range(CH // 16):
                    pk = eblk_v[r, pl.ds(v * 16, 16)]
                    dv = pk & 16383
                    inr = (dv >= lo) & (dv < lo + half)
                    plsc.store_compressed(cpk_v.at[pl.ds(cw, 16)], pk - lo,
                                          mask=inr)
                    cw = cw + jnp.sum(inr.astype(jnp.int32))
                return cw

            return lax.fori_loop(0, SB, frow, cw)

        cw = lax.fori_loop(0, ch_per_tile // SB, fblock, 0)

        for g2 in range(17):
            pidx = cw + g2 * 16 + iota
            plsc.store_scatter(cpk_v, [pidx], trash16i)

        def stage(j, p):
            for v in range(CH // 16):
                pk = cpk_v[pl.ds(j * CH + v * 16, 16)]
                lv = pk & 16383
                lsrc_v[p, pl.ds(v * 16, 16)] = pk >> 14
                ldst_v[p, pl.ds(v * 16, 16)] = lv
                fidx = copy_idx * hb + lv
                for g in range(4):
                    plsc.addupdate_scatter(deg4_v, [fidx], ones16,
                                           mask=group_masks[g])

        stage(0, 0)

        def body(j, _):
            p = j & 1
            cp = pltpu.async_copy(h_hbm.at[lsrc_v.at[p]], rows_v, semA)
            stage(j + 1, 1 - p)
            cp.wait()
            pltpu.sync_copy(rows_v, agg_sh.at[ldst_v.at[p]], add=True)
            return _

        nct = (cw + (CH - 1)) // CH
        lax.fori_loop(0, nct, body, None)

        def red(i, _):
            b = i * 16
            deg_v[pl.ds(b, 16)] = (
                (deg4_v[pl.ds(b, 16)] + deg4_v[pl.ds(hb + b, 16)])
                + (deg4_v[pl.ds(2 * hb + b, 16)]
                   + deg4_v[pl.ds(3 * hb + b, 16)]))
            return _

        lax.fori_loop(0, hb // 16, red, None)
        plsc.subcore_barrier()

        pltpu.sync_copy(agg_sh.at[pl.ds(r0, rows_per_tile)],
                        oagg_hbm.at[c, pl.ds(r0, rows_per_tile)])
        pltpu.sync_copy(deg_v, odeg_hbm.at[s, pl.ds(c * hb, hb)])

    return k(h, epk2, zeros_agg, zeros_deg)


def _tc_mlp(pagg, pdeg, h, W1, b1, W2, b2, gamma, beta, n, half, d, out_d):
    hb = half + TR
    n1 = n - half

    def body(pa_ref, pd_ref, h_ref, w1_ref, b1_ref, w2_ref, b2_ref,
             g_ref, be_ref, o_ref):
        agg = jnp.concatenate([pa_ref[0, :half, :], pa_ref[1, :n1, :]], axis=0)
        ones_col = jnp.ones((pd_ref.shape[0], 1), jnp.float32)
        deg_col = lax.dot_general(pd_ref[...], ones_col,
                                  dimension_numbers=(((0,), (0,)), ((), ())),
                                  preferred_element_type=jnp.float32)
        deg = jnp.concatenate([deg_col[:half, :], deg_col[hb:hb + n1, :]],
                              axis=0)
        h_in = agg / jnp.maximum(deg, 1.0) + h_ref[...]
        z = jnp.dot(h_in, w1_ref[...], preferred_element_type=jnp.float32)
        z = jnp.maximum(z + b1_ref[...], 0.0)
        z = jnp.dot(z, w2_ref[...], preferred_element_type=jnp.float32)
        z = jnp.maximum(z + b2_ref[...], 0.0)
        mean = jnp.mean(z, axis=0, keepdims=True)
        zc = z - mean
        var = jnp.mean(zc * zc, axis=0, keepdims=True)
        o_ref[...] = zc * lax.rsqrt(var + 1e-5) * g_ref[...] + be_ref[...]

    return pl.pallas_call(
        body,
        out_shape=jax.ShapeDtypeStruct((n, out_d), jnp.float32),
    )(pagg, pdeg, h, W1, b1, W2, b2, gamma, beta)


def kernel(h, edge_index, W1, b1, W2, b2, gamma, beta):
    n, d = h.shape
    e = edge_index.shape[1]
    hdim = W1.shape[1]
    out_d = W2.shape[1]

    half = ((n + 2 * 128 - 1) // (2 * 128)) * 128
    epg = CH * NS * 8
    e_pad = ((e + epg - 1) // epg) * epg
    pad = e_pad - e

    src = edge_index[0].astype(jnp.int32)
    dst = edge_index[1].astype(jnp.int32)
    assert n < (1 << 14) and half + TR < (1 << 14)
    if pad:
        src = jnp.concatenate([src, jnp.zeros((pad,), jnp.int32)])
        dst = jnp.concatenate([dst, jnp.full((pad,), n, jnp.int32)])
    epk2 = ((src << 14) | dst).reshape(-1, CH)

    hb = half + TR
    zeros_agg = jnp.zeros((hb, d), jnp.float32)
    zeros_deg = jnp.zeros((4 * hb,), jnp.float32)

    pagg, pdeg = _sc_scatter(h, epk2, zeros_agg, zeros_deg, half, d)
    return _tc_mlp(pagg, pdeg, h,
                   W1, b1.reshape(1, hdim), W2, b2.reshape(1, out_d),
                   gamma.reshape(1, out_d), beta.reshape(1, out_d),
                   n, half, d, out_d)
